# baseline (speedup 1.0000x reference)
import functools

import jax
import jax.numpy as jnp
from jax import lax
from jax.experimental import pallas as pl
from jax.experimental.pallas import tpu as pltpu
from jax.experimental.pallas import tpu_sc as plsc

B = 1024
CTX = 20
D = 64
V = 100000

NC = 2
NS = 16
NW = NC * NS
BPW = B // NW
IPW = BPW * CTX
CHUNK = 128
NCHUNK = IPW // CHUNK


DP = 128


def _pool_sc(xf, emb_p):
    mesh = plsc.VectorSubcoreMesh(
        core_axis_name="c", subcore_axis_name="s", num_cores=NC, num_subcores=NS
    )

    @functools.partial(
        pl.kernel,
        mesh=mesh,
        out_type=jax.ShapeDtypeStruct((B, DP), jnp.bfloat16),
        scratch_types=[
            pltpu.VMEM((IPW,), jnp.int32),
            pltpu.VMEM((IPW, DP), jnp.bfloat16),
            pltpu.VMEM((BPW, DP), jnp.bfloat16),
            pltpu.SemaphoreType.DMA,
        ],
        compiler_params=pltpu.CompilerParams(use_tc_tiling_on_sc=False, needs_layout_passes=False),
    )
    def pool(xf_hbm, emb_hbm, out_hbm, idx_v, rows_v, m_v, sem):
        wid = lax.axis_index("s") * NC + lax.axis_index("c")
        base = wid * IPW
        pltpu.sync_copy(xf_hbm.at[pl.ds(base, IPW)], idx_v)
        copies = [
            pltpu.async_copy(
                emb_hbm.at[idx_v.at[pl.ds(j * CHUNK, CHUNK)]],
                rows_v.at[pl.ds(j * CHUNK, CHUNK)],
                sem,
            )
            for j in range(NCHUNK)
        ]
        for c in copies:
            c.wait()

        def body(i, carry):
            for d in range(D // 32):
                a0, a1 = plsc.unpack(
                    rows_v[i * CTX, pl.ds(d * 32, 32)],
                    format=plsc.PackFormat.INTERLEAVED,
                )
                for t in range(1, CTX):
                    b0, b1 = plsc.unpack(
                        rows_v[i * CTX + t, pl.ds(d * 32, 32)],
                        format=plsc.PackFormat.INTERLEAVED,
                    )
                    a0 = a0 + b0
                    a1 = a1 + b1
                m_v[i, pl.ds(d * 32, 32)] = plsc.pack(
                    a0 * (1.0 / CTX),
                    a1 * (1.0 / CTX),
                    format=plsc.PackFormat.INTERLEAVED,
                )
            return carry

        lax.fori_loop(0, BPW, body, 0)
        pltpu.sync_copy(m_v, out_hbm.at[pl.ds(wid * BPW, BPW)])

    return pool(xf, emb_p)


VB = 4096


VR = 16384


def _repack_tc(embT):
    def rpk(x_ref, o_ref):
        o_ref[:, :D] = x_ref[...].T.astype(jnp.bfloat16)
        o_ref[:, D:] = jnp.zeros((VR, DP - D), jnp.bfloat16)

    return pl.pallas_call(
        rpk,
        grid=(pl.cdiv(V, VR),),
        in_specs=[pl.BlockSpec((D, VR), lambda j: (0, j))],
        out_specs=pl.BlockSpec((VR, DP), lambda j: (j, 0)),
        out_shape=jax.ShapeDtypeStruct((V, DP), jnp.bfloat16),
        compiler_params=pltpu.CompilerParams(dimension_semantics=("parallel",)),
    )(embT)


def _matmul_tc(m, W, b2):
    def mmk(wt_ref, m_ref, b_ref, o_ref):
        o_ref[...] = (
            lax.dot_general(
                wt_ref[...].astype(jnp.bfloat16),
                m_ref[:, :D],
                (((0,), (1,)), ((), ())),
                preferred_element_type=jnp.float32,
            )
            + b_ref[...][:, None]
        )

    return pl.pallas_call(
        mmk,
        grid=(pl.cdiv(V, VB),),
        in_specs=[
            pl.BlockSpec((D, VB), lambda j: (0, j)),
            pl.BlockSpec((B, DP), lambda j: (0, 0)),
            pl.BlockSpec((VB,), lambda j: (j,)),
        ],
        out_specs=pl.BlockSpec((VB, B), lambda j: (j, 0)),
        out_shape=jax.ShapeDtypeStruct((V, B), jnp.float32),
        compiler_params=pltpu.CompilerParams(
            fuse_transposed_lhs_in_matmul=True,
            dimension_semantics=("parallel",),
        ),
    )(W, m, b2)


def kernel(x, emb, W, b):
    xf = x.reshape(-1)
    emb_p = _repack_tc(emb.T)
    m = _pool_sc(xf, emb_p)
    return _matmul_tc(m, W.T, b).T


# speedup vs baseline: 1.3494x; 1.3494x over previous
import functools

import jax
import jax.numpy as jnp
from jax import lax
from jax.experimental import pallas as pl
from jax.experimental.pallas import tpu as pltpu
from jax.experimental.pallas import tpu_sc as plsc

B = 1024
CTX = 20
D = 64
V = 100000

NC = 2
NS = 16
NW = NC * NS
B2 = B // 2
BPW = B2 // NW
IPW = BPW * CTX
CHUNK = 80
NCHUNK = IPW // CHUNK


DP = 128


def _pool_sc(xf, emb_p, half):
    mesh = plsc.VectorSubcoreMesh(
        core_axis_name="c", subcore_axis_name="s", num_cores=NC, num_subcores=NS
    )

    @functools.partial(
        pl.kernel,
        mesh=mesh,
        out_type=jax.ShapeDtypeStruct((B2, DP), jnp.float32),
        scratch_types=[
            pltpu.VMEM((IPW,), jnp.int32),
            pltpu.VMEM((IPW, DP), jnp.float32),
            pltpu.VMEM((BPW, DP), jnp.float32),
            pltpu.SemaphoreType.DMA,
        ],
        compiler_params=pltpu.CompilerParams(use_tc_tiling_on_sc=True),
    )
    def pool(xf_hbm, emb_hbm, out_hbm, idx_v, rows_v, m_v, sem):
        wid = lax.axis_index("s") * NC + lax.axis_index("c")
        base = half * (B2 * CTX) + wid * IPW
        pltpu.sync_copy(xf_hbm.at[pl.ds(base, IPW)], idx_v)
        copies = [
            pltpu.async_copy(
                emb_hbm.at[idx_v.at[pl.ds(j * CHUNK, CHUNK)]],
                rows_v.at[pl.ds(j * CHUNK, CHUNK)],
                sem,
            )
            for j in range(NCHUNK)
        ]
        for c in copies:
            c.wait()

        def body(i, carry):
            for d in range(D // 16):
                acc = rows_v[i * CTX, pl.ds(d * 16, 16)]
                for t in range(1, CTX):
                    acc = acc + rows_v[i * CTX + t, pl.ds(d * 16, 16)]
                m_v[i, pl.ds(d * 16, 16)] = acc * (1.0 / CTX)
            return carry

        lax.fori_loop(0, BPW, body, 0)
        pltpu.sync_copy(m_v, out_hbm.at[pl.ds(wid * BPW, BPW)])

    return pool(xf, emb_p)


def _mm_half(WT, mh, b, half, prev=None):
    def mmk(wt_ref, m_ref, b_ref, *rest):
        o_ref = rest[-1]
        o_ref[...] = (
            lax.dot_general(
                wt_ref[...],
                m_ref[:, :D],
                (((0,), (1,)), ((), ())),
                preferred_element_type=jnp.float32,
            )
            + b_ref[...][:, None]
        )

    in_specs = [
        pl.BlockSpec((D, VB), lambda j: (0, j)),
        pl.BlockSpec((B2, DP), lambda j: (0, 0)),
        pl.BlockSpec((VB,), lambda j: (j,)),
    ]
    args = [WT, mh, b]
    aliases = {}
    if prev is not None:
        in_specs.append(pl.BlockSpec(memory_space=pl.ANY))
        args.append(prev)
        aliases = {3: 0}
    return pl.pallas_call(
        mmk,
        grid=(pl.cdiv(V, VB),),
        in_specs=in_specs,
        out_specs=pl.BlockSpec((VB, B2), lambda j, h=half: (j, h)),
        out_shape=jax.ShapeDtypeStruct((V, B), jnp.float32),
        input_output_aliases=aliases,
        compiler_params=pltpu.CompilerParams(
            fuse_transposed_lhs_in_matmul=True,
            dimension_semantics=("parallel",),
        ),
    )(*args)


VB = 4096


VR = 16384


def _repack_tc(embT):
    def rpk(x_ref, o_ref):
        o_ref[:, :D] = x_ref[...].T
        o_ref[:, D:] = jnp.zeros((VR, DP - D), jnp.float32)

    return pl.pallas_call(
        rpk,
        grid=(pl.cdiv(V, VR),),
        in_specs=[pl.BlockSpec((D, VR), lambda j: (0, j))],
        out_specs=pl.BlockSpec((VR, DP), lambda j: (j, 0)),
        out_shape=jax.ShapeDtypeStruct((V, DP), jnp.float32),
        compiler_params=pltpu.CompilerParams(dimension_semantics=("parallel",)),
    )(embT)


def _matmul_tc(m, W, b2):
    def mmk(wt_ref, m_ref, b_ref, o_ref):
        o_ref[...] = (
            lax.dot_general(
                wt_ref[...],
                m_ref[:, :D],
                (((0,), (1,)), ((), ())),
                preferred_element_type=jnp.float32,
            )
            + b_ref[...][:, None]
        )

    return pl.pallas_call(
        mmk,
        grid=(pl.cdiv(V, VB),),
        in_specs=[
            pl.BlockSpec((D, VB), lambda j: (0, j)),
            pl.BlockSpec((B, DP), lambda j: (0, 0)),
            pl.BlockSpec((VB,), lambda j: (j,)),
        ],
        out_specs=pl.BlockSpec((VB, B), lambda j: (j, 0)),
        out_shape=jax.ShapeDtypeStruct((V, B), jnp.float32),
        compiler_params=pltpu.CompilerParams(
            fuse_transposed_lhs_in_matmul=True,
            dimension_semantics=("parallel",),
        ),
    )(W, m, b2)


def kernel(x, emb, W, b):
    xf = x.reshape(-1)
    emb_p = _repack_tc(emb.T)
    m0 = _pool_sc(xf, emb_p, 0)
    m1 = _pool_sc(xf, emb_p, 1)
    o0 = _mm_half(W.T, m0, b, 0)
    return _mm_half(W.T, m1, b, 1, prev=o0).T


# speedup vs baseline: 1.4656x; 1.0861x over previous
import functools

import jax
import jax.numpy as jnp
from jax import lax
from jax.experimental import pallas as pl
from jax.experimental.pallas import tpu as pltpu
from jax.experimental.pallas import tpu_sc as plsc

B = 1024
CTX = 20
D = 64
V = 100000

NC = 2
NS = 16
NW = NC * NS
BPW = B // NW
IPW = BPW * CTX
CHUNK = 128
NCHUNK = IPW // CHUNK


DP = 128


def _pool_sc(xf, emb_p):
    mesh = plsc.VectorSubcoreMesh(
        core_axis_name="c", subcore_axis_name="s", num_cores=NC, num_subcores=NS
    )

    @functools.partial(
        pl.kernel,
        mesh=mesh,
        out_type=jax.ShapeDtypeStruct((B, DP), jnp.float32),
        scratch_types=[
            pltpu.VMEM((IPW,), jnp.int32),
            pltpu.VMEM((IPW, D), jnp.float32),
            pltpu.VMEM((BPW, DP), jnp.float32),
            pltpu.SemaphoreType.DMA,
        ],
        compiler_params=pltpu.CompilerParams(use_tc_tiling_on_sc=False),
    )
    def pool(xf_hbm, emb_hbm, out_hbm, idx_v, rows_v, m_v, sem):
        wid = lax.axis_index("s") * NC + lax.axis_index("c")
        base = wid * IPW
        pltpu.sync_copy(xf_hbm.at[pl.ds(base, IPW)], idx_v)
        copies = [
            pltpu.async_copy(
                emb_hbm.at[idx_v.at[pl.ds(j * CHUNK, CHUNK)]],
                rows_v.at[pl.ds(j * CHUNK, CHUNK)],
                sem,
            )
            for j in range(NCHUNK)
        ]
        for c in copies:
            c.wait()

        def body(i, carry):
            for d in range(D // 16):
                acc = rows_v[i * CTX, pl.ds(d * 16, 16)]
                for t in range(1, CTX):
                    acc = acc + rows_v[i * CTX + t, pl.ds(d * 16, 16)]
                m_v[i, pl.ds(d * 16, 16)] = acc * (1.0 / CTX)
            return carry

        lax.fori_loop(0, BPW, body, 0)
        pltpu.sync_copy(m_v, out_hbm.at[pl.ds(wid * BPW, BPW)])

    return pool(xf, emb_p)


VB = 4096


VR = 16384


def _repack_tc(embT):
    def rpk(x_ref, o_ref):
        o_ref[:, :D] = x_ref[...].T
        o_ref[:, D:] = jnp.zeros((VR, DP - D), jnp.float32)

    return pl.pallas_call(
        rpk,
        grid=(pl.cdiv(V, VR),),
        in_specs=[pl.BlockSpec((D, VR), lambda j: (0, j))],
        out_specs=pl.BlockSpec((VR, DP), lambda j: (j, 0)),
        out_shape=jax.ShapeDtypeStruct((V, DP), jnp.float32),
        compiler_params=pltpu.CompilerParams(dimension_semantics=("parallel",)),
    )(embT)


def _matmul_tc(m, W, b2):
    def mmk(wt_ref, m_ref, b_ref, o_ref):
        o_ref[...] = (
            lax.dot_general(
                wt_ref[...],
                m_ref[:, :D],
                (((0,), (1,)), ((), ())),
                preferred_element_type=jnp.float32,
            )
            + b_ref[...][:, None]
        )

    return pl.pallas_call(
        mmk,
        grid=(pl.cdiv(V, VB),),
        in_specs=[
            pl.BlockSpec((D, VB), lambda j: (0, j)),
            pl.BlockSpec((B, DP), lambda j: (0, 0)),
            pl.BlockSpec((VB,), lambda j: (j,)),
        ],
        out_specs=pl.BlockSpec((VB, B), lambda j: (j, 0)),
        out_shape=jax.ShapeDtypeStruct((V, B), jnp.float32),
        compiler_params=pltpu.CompilerParams(
            fuse_transposed_lhs_in_matmul=True,
            dimension_semantics=("parallel",),
        ),
    )(W, m, b2)


def kernel(x, emb, W, b):
    xf = x.reshape(-1) * 2
    emb_p = _repack_tc(emb.T)
    m = _pool_sc(xf, emb_p.reshape(2 * V, D))
    return _matmul_tc(m, W.T, b).T
